# baseline (speedup 1.0000x reference)
import functools

import jax
import jax.numpy as jnp
from jax import lax
from jax.experimental import pallas as pl
from jax.experimental.pallas import tpu as pltpu
from jax.experimental.pallas import tpu_sc as plsc

B = 16384
NUM_ROWS = 2000000
ED = 16
TOT = 2 * B
NC, NS = 2, 16
NW = NC * NS
PER_W = TOT // NW
CALLS = PER_W * ED // 128
GROUPS = 8
GSZ = CALLS // GROUPS
OUTW = TOT // 8


def _gather_body(widx_hbm, tv_hbm, out_hbm, idx_v, rows_v, sem):
    wid = lax.axis_index("s") * NC + lax.axis_index("c")
    pltpu.sync_copy(widx_hbm.at[pl.ds(wid * CALLS, CALLS)], idx_v)

    def group(g, carry):
        for j in range(GSZ):
            pltpu.async_copy(tv_hbm.at[idx_v.at[g * GSZ + j]],
                             rows_v.at[g * GSZ + j], sem)
        return carry

    lax.fori_loop(0, GROUPS, group, 0)
    pltpu.make_async_copy(out_hbm.at[pl.ds(0, CALLS)], rows_v, sem).wait()
    pltpu.sync_copy(rows_v, out_hbm.at[pl.ds(wid * CALLS, CALLS)])


_gather = functools.partial(
    pl.kernel,
    out_type=jax.ShapeDtypeStruct((OUTW, 128), jnp.float32),
    mesh=plsc.VectorSubcoreMesh(core_axis_name="c", subcore_axis_name="s"),
    scratch_types=[
        pltpu.VMEM((CALLS, 128), jnp.int32),
        pltpu.VMEM((CALLS, 128), jnp.float32),
        pltpu.SemaphoreType.DMA,
    ],
)(_gather_body)


def _mlp_body(uw_ref, vw_ref, W1u_ref, W1v_ref, b1w_ref, W2w_ref, b2w_ref,
              W3w_ref, b3w_ref, Wg_ref, Wh_ref, bfc_ref, out_ref):
    uw = uw_ref[...]
    vw = vw_ref[...]
    f32 = jnp.float32
    h = jnp.maximum(jnp.dot(uw, W1u_ref[...], preferred_element_type=f32)
                    + jnp.dot(vw, W1v_ref[...], preferred_element_type=f32)
                    + b1w_ref[...], 0.0)
    h = jnp.maximum(jnp.dot(h, W2w_ref[...], preferred_element_type=f32)
                    + b2w_ref[...], 0.0)
    h = jnp.maximum(jnp.dot(h, W3w_ref[...], preferred_element_type=f32)
                    + b3w_ref[...], 0.0)
    g = uw * vw
    out_ref[...] = (jnp.dot(g, Wg_ref[...], preferred_element_type=f32)
                    + jnp.dot(h, Wh_ref[...], preferred_element_type=f32)
                    + bfc_ref[...])


def kernel(x, table, W1, b1, W2, b2, W3, b3, Wfc, bfc):
    xi = x.astype(jnp.int32)
    iflat = jnp.concatenate([xi[:, 0], xi[:, 1] + (NUM_ROWS // 2)])
    base = (iflat >> 7) * 1024 + (iflat & 127)
    d = jnp.arange(ED, dtype=jnp.int32)
    dofs = (d >> 3) * (NUM_ROWS * 8) + (d & 7) * 128
    widx = (base[:, None] + dofs[None, :]).reshape(NW * CALLS, 128)

    tv = table.T.reshape(2, 8, NUM_ROWS // 128, 128).transpose(0, 2, 1, 3).reshape(-1)

    wide = _gather(widx, tv)

    eye8 = jnp.eye(8, dtype=jnp.float32)
    W1u = (eye8[:, None, :, None] * W1[:ED][None, :, None, :]).reshape(128, 512)
    W1v = (eye8[:, None, :, None] * W1[ED:][None, :, None, :]).reshape(128, 512)
    b1w = jnp.tile(b1, 8).reshape(1, 512)
    W2w = (eye8[:, None, :, None] * W2[None, :, None, :]).reshape(512, 256)
    b2w = jnp.tile(b2, 8).reshape(1, 256)
    W3w = (eye8[:, None, :, None] * W3[None, :, None, :]).reshape(256, 128)
    b3w = jnp.tile(b3, 8).reshape(1, 128)
    Wg = (eye8[:, None, :] * Wfc[:ED, 0][None, :, None]).reshape(128, 8)
    Wh = (eye8[:, None, :] * Wfc[ED:, 0][None, :, None]).reshape(128, 8)

    BK8 = 2048
    grid = B // (BK8 * 8)
    out = pl.pallas_call(
        _mlp_body,
        grid=(grid,),
        in_specs=[pl.BlockSpec((BK8, 128), lambda i: (i, 0)),
                  pl.BlockSpec((BK8, 128), lambda i, g=grid: (i + g, 0))]
        + [pl.BlockSpec(s, (lambda i: (0,) * n))
           for s, n in [((128, 512), 2), ((128, 512), 2), ((1, 512), 2),
                        ((512, 256), 2), ((1, 256), 2), ((256, 128), 2),
                        ((1, 128), 2), ((128, 8), 2), ((128, 8), 2),
                        ((1, 1), 2)]],
        out_specs=pl.BlockSpec((BK8, 8), lambda i: (i, 0)),
        out_shape=jax.ShapeDtypeStruct((B // 8, 8), jnp.float32),
    )(wide, wide, W1u, W1v, b1w, W2w, b2w, W3w, b3w, Wg, Wh,
      bfc.reshape(1, 1))
    return out.reshape(-1)


# speedup vs baseline: 1.0067x; 1.0067x over previous
import functools

import jax
import jax.numpy as jnp
from jax import lax
from jax.experimental import pallas as pl
from jax.experimental.pallas import tpu as pltpu
from jax.experimental.pallas import tpu_sc as plsc

B = 16384
NUM_ROWS = 2000000
ED = 16
TOT = 2 * B
NC, NS = 2, 16
NW = NC * NS
PER_W = TOT // NW
CALLS = PER_W * ED // 128
GROUPS = 1
GSZ = CALLS // GROUPS
OUTW = TOT // 8


def _gather_body(widx_hbm, tv_hbm, out_hbm, idx_v, rows_v, sem):
    wid = lax.axis_index("s") * NC + lax.axis_index("c")
    pltpu.sync_copy(widx_hbm.at[pl.ds(wid * CALLS, CALLS)], idx_v)

    def group(g, carry):
        for j in range(GSZ):
            pltpu.async_copy(tv_hbm.at[idx_v.at[g * GSZ + j]],
                             rows_v.at[g * GSZ + j], sem)
        return carry

    lax.fori_loop(0, GROUPS, group, 0)
    pltpu.make_async_copy(out_hbm.at[pl.ds(0, CALLS)], rows_v, sem).wait()
    pltpu.sync_copy(rows_v, out_hbm.at[pl.ds(wid * CALLS, CALLS)])


_gather = functools.partial(
    pl.kernel,
    out_type=jax.ShapeDtypeStruct((OUTW, 128), jnp.float32),
    mesh=plsc.VectorSubcoreMesh(core_axis_name="c", subcore_axis_name="s"),
    scratch_types=[
        pltpu.VMEM((CALLS, 128), jnp.int32),
        pltpu.VMEM((CALLS, 128), jnp.float32),
        pltpu.SemaphoreType.DMA,
    ],
)(_gather_body)


def _mlp_body(uw_ref, vw_ref, W1u_ref, W1v_ref, b1w_ref, W2w_ref, b2w_ref,
              W3w_ref, b3w_ref, Wg_ref, Wh_ref, bfc_ref, out_ref):
    uw = uw_ref[...]
    vw = vw_ref[...]
    f32 = jnp.float32
    h = jnp.maximum(jnp.dot(uw, W1u_ref[...], preferred_element_type=f32)
                    + jnp.dot(vw, W1v_ref[...], preferred_element_type=f32)
                    + b1w_ref[...], 0.0)
    h = jnp.maximum(jnp.dot(h, W2w_ref[...], preferred_element_type=f32)
                    + b2w_ref[...], 0.0)
    h = jnp.maximum(jnp.dot(h, W3w_ref[...], preferred_element_type=f32)
                    + b3w_ref[...], 0.0)
    g = uw * vw
    out_ref[...] = (jnp.dot(g, Wg_ref[...], preferred_element_type=f32)
                    + jnp.dot(h, Wh_ref[...], preferred_element_type=f32)
                    + bfc_ref[...])


def kernel(x, table, W1, b1, W2, b2, W3, b3, Wfc, bfc):
    offsets = jnp.array([0, NUM_ROWS // 2], dtype=x.dtype)
    iflat = (x + offsets[None, :]).astype(jnp.int32).T.reshape(-1)
    base = (iflat >> 7) * 1024 + (iflat & 127)
    d = jnp.arange(ED, dtype=jnp.int32)
    dofs = (d >> 3) * (NUM_ROWS * 8) + (d & 7) * 128
    widx = (base[:, None] + dofs[None, :]).reshape(NW * CALLS, 128)

    tv = table.T.reshape(2, 8, NUM_ROWS // 128, 128).transpose(0, 2, 1, 3).reshape(-1)

    wide = _gather(widx, tv)

    eye8 = jnp.eye(8, dtype=jnp.float32)
    W1u = (eye8[:, None, :, None] * W1[:ED][None, :, None, :]).reshape(128, 512)
    W1v = (eye8[:, None, :, None] * W1[ED:][None, :, None, :]).reshape(128, 512)
    b1w = jnp.tile(b1, 8).reshape(1, 512)
    W2w = (eye8[:, None, :, None] * W2[None, :, None, :]).reshape(512, 256)
    b2w = jnp.tile(b2, 8).reshape(1, 256)
    W3w = (eye8[:, None, :, None] * W3[None, :, None, :]).reshape(256, 128)
    b3w = jnp.tile(b3, 8).reshape(1, 128)
    Wg = (eye8[:, None, :] * Wfc[:ED, 0][None, :, None]).reshape(128, 8)
    Wh = (eye8[:, None, :] * Wfc[ED:, 0][None, :, None]).reshape(128, 8)

    BK8 = 2048
    grid = B // (BK8 * 8)
    out = pl.pallas_call(
        _mlp_body,
        grid=(grid,),
        in_specs=[pl.BlockSpec((BK8, 128), lambda i: (i, 0)),
                  pl.BlockSpec((BK8, 128), lambda i, g=grid: (i + g, 0))]
        + [pl.BlockSpec(s, (lambda i: (0,) * n))
           for s, n in [((128, 512), 2), ((128, 512), 2), ((1, 512), 2),
                        ((512, 256), 2), ((1, 256), 2), ((256, 128), 2),
                        ((1, 128), 2), ((128, 8), 2), ((128, 8), 2),
                        ((1, 1), 2)]],
        out_specs=pl.BlockSpec((BK8, 8), lambda i: (i, 0)),
        out_shape=jax.ShapeDtypeStruct((B // 8, 8), jnp.float32),
    )(wide, wide, W1u, W1v, b1w, W2w, b2w, W3w, b3w, Wg, Wh,
      bfc.reshape(1, 1))
    return out.reshape(-1)
